# baseline (speedup 1.0000x reference)
import functools

import jax
import jax.numpy as jnp
from jax import lax
from jax.experimental import pallas as pl
from jax.experimental.pallas import tpu as pltpu
from jax.experimental.pallas import tpu_sc as plsc

VOCAB = 1000000
B = 16384
D = 64
NNEG = 20
L = 16
NC, NS = 2, 16
NW = NC * NS
BPW = B // NW
CHUNK = 32
NCHUNK = BPW // CHUNK
GPC = CHUNK // L
NEG_ROWS = CHUNK * NNEG
NIDX_W = 128
NIDX_ROWS = NEG_ROWS // NIDX_W
WIDXR = NCHUNK * NIDX_ROWS
QPW = BPW // NIDX_W
RB = 8000


def _sc_scores(cw, xw, neg2d, w2_center, w2_context):
    mesh = plsc.VectorSubcoreMesh(core_axis_name="c", subcore_axis_name="s")

    @functools.partial(
        pl.kernel,
        mesh=mesh,
        out_type=[
            jax.ShapeDtypeStruct((B,), jnp.float32),
            jax.ShapeDtypeStruct((NW * NNEG * QPW, NIDX_W), jnp.float32),
        ],
        scratch_types=[
            pltpu.VMEM((BPW,), jnp.int32),
            pltpu.VMEM((BPW,), jnp.int32),
            pltpu.VMEM((WIDXR, NIDX_W), jnp.int32),
            pltpu.VMEM((CHUNK, D), jnp.float32),
            pltpu.VMEM((CHUNK, D), jnp.float32),
            pltpu.VMEM((NEG_ROWS, D), jnp.float32),
            pltpu.VMEM((CHUNK, D), jnp.float32),
            pltpu.VMEM((CHUNK, D), jnp.float32),
            pltpu.VMEM((NEG_ROWS, D), jnp.float32),
            pltpu.VMEM((BPW,), jnp.float32),
            pltpu.VMEM((NNEG * QPW, NIDX_W), jnp.float32),
            pltpu.SemaphoreType.DMA,
            pltpu.SemaphoreType.DMA,
        ],
        compiler_params=pltpu.CompilerParams(
            needs_layout_passes=False, use_tc_tiling_on_sc=False),
    )
    def body(cw_hbm, xw_hbm, neg_hbm, wc_hbm, wx_hbm, pos_out, negt_out,
             idx_c, idx_x, idx_n,
             rows_c0, rows_x0, rows_n0, rows_c1, rows_x1, rows_n1,
             pos_buf, negt_buf, sem0, sem1):
        wid = lax.axis_index("s") * NC + lax.axis_index("c")
        base = wid * BPW
        lane = lax.iota(jnp.int32, L)
        bufs = ((rows_c0, rows_x0, rows_n0, sem0),
                (rows_c1, rows_x1, rows_n1, sem1))

        pltpu.sync_copy(cw_hbm.at[pl.ds(base, BPW)], idx_c)
        pltpu.sync_copy(xw_hbm.at[pl.ds(base, BPW)], idx_x)
        nbase = pl.multiple_of(wid * WIDXR, 8)
        pltpu.sync_copy(neg_hbm.at[pl.ds(nbase, WIDXR)], idx_n)

        def chunk_dmas(ci, b):
            rows_c, rows_x, rows_n, sem = bufs[b]
            cps = [
                pltpu.make_async_copy(
                    wc_hbm.at[idx_c.at[pl.ds(ci * CHUNK, CHUNK)]],
                    rows_c, sem),
                pltpu.make_async_copy(
                    wx_hbm.at[idx_x.at[pl.ds(ci * CHUNK, CHUNK)]],
                    rows_x, sem),
            ]
            for j in range(NIDX_ROWS):
                cps.append(pltpu.make_async_copy(
                    wx_hbm.at[idx_n.at[ci * NIDX_ROWS + j]],
                    rows_n.at[pl.ds(j * NIDX_W, NIDX_W)], sem))
            return cps

        def fire(ci, b):
            for cp in chunk_dmas(ci, b):
                cp.start()

        def drain(ci, b):
            for cp in chunk_dmas(ci, b):
                cp.wait()

        def compute(ci, b):
            rows_c, rows_x, rows_n, _ = bufs[b]

            def group_body(g, gcarry):
                row16 = g * L + lane

                def d_body(it, accs):
                    d0 = it * 8
                    new = list(accs)
                    for u in range(8):
                        rot = (lane + (d0 + u)) & (D - 1)
                        cv = plsc.load_gather(rows_c, [row16, rot])
                        xv = plsc.load_gather(rows_x, [row16, rot])
                        new[0] = new[0] + cv * xv
                        for n in range(NNEG):
                            nv = plsc.load_gather(
                                rows_n, [row16 + n * CHUNK, rot])
                            new[n + 1] = new[n + 1] + cv * nv
                    return tuple(new)

                zero = jnp.zeros((L,), jnp.float32)
                accs = lax.fori_loop(0, D // 8, d_body, (zero,) * (NNEG + 1))
                off = ci * CHUNK + g * L
                q = off >> 7
                cq = off & (NIDX_W - 1)
                pos_buf[pl.ds(off, L)] = accs[0]
                for n in range(NNEG):
                    negt_buf[n * QPW + q, pl.ds(cq, L)] = accs[n + 1]
                return gcarry

            lax.fori_loop(0, GPC, group_body, 0)

        fire(0, 0)

        def pair_body(p, carry):
            e = 2 * p
            drain(e, 0)
            fire(e + 1, 1)
            compute(e, 0)
            drain(e + 1, 1)

            @pl.when(p < NCHUNK // 2 - 1)
            def _prefetch():
                fire(e + 2, 0)

            compute(e + 1, 1)
            return carry

        lax.fori_loop(0, NCHUNK // 2, pair_body, 0)
        pltpu.sync_copy(pos_buf, pos_out.at[pl.ds(base, BPW)])
        obase = pl.multiple_of(wid * (NNEG * QPW), 8)
        pltpu.sync_copy(negt_buf, negt_out.at[pl.ds(obase, NNEG * QPW)])

    return body(cw, xw, neg2d, w2_center, w2_context)


def _tc_loss(pos2d, negflat):
    def body(pos_ref, neg_ref, out_ref):
        def log_sigmoid(x):
            return jnp.minimum(x, 0.0) - jnp.log(1.0 + jnp.exp(-jnp.abs(x)))
        s = jnp.sum(log_sigmoid(pos_ref[...])) \
            + jnp.sum(log_sigmoid(-neg_ref[...]))
        out_ref[0, 0] = -s / B

    return pl.pallas_call(
        body,
        out_shape=jax.ShapeDtypeStruct((1, 1), jnp.float32),
        out_specs=pl.BlockSpec(memory_space=pltpu.SMEM),
    )(pos2d, negflat)


def kernel(center_words, context_words, negative_words, W_center, W_context):
    cw = center_words.astype(jnp.int32)
    xw = context_words.astype(jnp.int32)
    neg2d = (negative_words.astype(jnp.int32)
             .reshape(NW, NCHUNK, CHUNK, NNEG)
             .transpose(0, 1, 3, 2)
             .reshape(B * NNEG // NIDX_W, NIDX_W))
    pos, negt = _sc_scores(cw, xw, neg2d, W_center, W_context)
    loss = _tc_loss(pos.reshape(B // NIDX_W, NIDX_W), negt)
    return loss[0, 0]


# speedup vs baseline: 1.0528x; 1.0528x over previous
import functools

import jax
import jax.numpy as jnp
from jax import lax
from jax.experimental import pallas as pl
from jax.experimental.pallas import tpu as pltpu
from jax.experimental.pallas import tpu_sc as plsc

VOCAB = 1000000
B = 16384
D = 64
NNEG = 20
L = 16
NC, NS = 2, 16
NW = NC * NS
BPW = B // NW
CHUNK = 32
NCHUNK = BPW // CHUNK
GPC = CHUNK // L
NEG_ROWS = CHUNK * NNEG
NIDX_W = 128
NIDX_ROWS = NEG_ROWS // NIDX_W
WIDXR = NCHUNK * NIDX_ROWS
QPW = BPW // NIDX_W
RB = 8000


def _sc_scores(cw, xw, neg2d, w2_center, w2_context):
    mesh = plsc.VectorSubcoreMesh(core_axis_name="c", subcore_axis_name="s")

    @functools.partial(
        pl.kernel,
        mesh=mesh,
        out_type=[
            jax.ShapeDtypeStruct((B,), jnp.float32),
            jax.ShapeDtypeStruct((NW * NNEG * QPW, NIDX_W), jnp.float32),
        ],
        scratch_types=[
            pltpu.VMEM((BPW,), jnp.int32),
            pltpu.VMEM((BPW,), jnp.int32),
            pltpu.VMEM((WIDXR, NIDX_W), jnp.int32),
            pltpu.VMEM((CHUNK, D), jnp.float32),
            pltpu.VMEM((CHUNK, D), jnp.float32),
            pltpu.VMEM((NEG_ROWS, D), jnp.float32),
            pltpu.VMEM((CHUNK, D), jnp.float32),
            pltpu.VMEM((CHUNK, D), jnp.float32),
            pltpu.VMEM((NEG_ROWS, D), jnp.float32),
            pltpu.VMEM((BPW,), jnp.float32),
            pltpu.VMEM((NNEG * QPW, NIDX_W), jnp.float32),
            pltpu.SemaphoreType.DMA,
            pltpu.SemaphoreType.DMA,
        ],
        compiler_params=pltpu.CompilerParams(
            needs_layout_passes=False, use_tc_tiling_on_sc=False),
    )
    def body(cw_hbm, xw_hbm, neg_hbm, wc_hbm, wx_hbm, pos_out, negt_out,
             idx_c, idx_x, idx_n,
             rows_c0, rows_x0, rows_n0, rows_c1, rows_x1, rows_n1,
             pos_buf, negt_buf, sem0, sem1):
        wid = lax.axis_index("s") * NC + lax.axis_index("c")
        base = wid * BPW
        lane = lax.iota(jnp.int32, L)
        bufs = ((rows_c0, rows_x0, rows_n0, sem0),
                (rows_c1, rows_x1, rows_n1, sem1))

        pltpu.sync_copy(cw_hbm.at[pl.ds(base, BPW)], idx_c)
        pltpu.sync_copy(xw_hbm.at[pl.ds(base, BPW)], idx_x)
        nbase = pl.multiple_of(wid * WIDXR, 8)
        pltpu.sync_copy(neg_hbm.at[pl.ds(nbase, WIDXR)], idx_n)

        def chunk_dmas(ci, b):
            rows_c, rows_x, rows_n, sem = bufs[b]
            cps = [
                pltpu.make_async_copy(
                    wc_hbm.at[idx_c.at[pl.ds(ci * CHUNK, CHUNK)]],
                    rows_c, sem),
                pltpu.make_async_copy(
                    wx_hbm.at[idx_x.at[pl.ds(ci * CHUNK, CHUNK)]],
                    rows_x, sem),
            ]
            for j in range(NIDX_ROWS):
                cps.append(pltpu.make_async_copy(
                    wx_hbm.at[idx_n.at[ci * NIDX_ROWS + j]],
                    rows_n.at[pl.ds(j * NIDX_W, NIDX_W)], sem))
            return cps

        def fire(ci, b):
            for cp in chunk_dmas(ci, b):
                cp.start()

        def drain(ci, b):
            for cp in chunk_dmas(ci, b):
                cp.wait()

        def compute(ci, b):
            rows_c, rows_x, rows_n, _ = bufs[b]

            def group_body(g, gcarry):
                row16 = g * L + lane

                def d_body(it, accs):
                    d0 = it * 4
                    new = list(accs)
                    for u in range(4):
                        rot = (lane + (d0 + u)) & (D - 1)
                        cv = plsc.load_gather(rows_c, [row16, rot])
                        xv = plsc.load_gather(rows_x, [row16, rot])
                        new[0] = new[0] + cv * xv
                        for n in range(NNEG):
                            nv = plsc.load_gather(
                                rows_n, [row16 + n * CHUNK, rot])
                            new[n + 1] = new[n + 1] + cv * nv
                    return tuple(new)

                zero = jnp.zeros((L,), jnp.float32)
                accs = lax.fori_loop(0, D // 4, d_body, (zero,) * (NNEG + 1))
                off = ci * CHUNK + g * L
                q = off >> 7
                cq = off & (NIDX_W - 1)
                pos_buf[pl.ds(off, L)] = accs[0]
                for n in range(NNEG):
                    negt_buf[n * QPW + q, pl.ds(cq, L)] = accs[n + 1]
                return gcarry

            lax.fori_loop(0, GPC, group_body, 0)

        fire(0, 0)

        def pair_body(p, carry):
            e = 2 * p
            drain(e, 0)
            fire(e + 1, 1)
            compute(e, 0)
            drain(e + 1, 1)

            @pl.when(p < NCHUNK // 2 - 1)
            def _prefetch():
                fire(e + 2, 0)

            compute(e + 1, 1)
            return carry

        lax.fori_loop(0, NCHUNK // 2, pair_body, 0)
        pltpu.sync_copy(pos_buf, pos_out.at[pl.ds(base, BPW)])
        obase = pl.multiple_of(wid * (NNEG * QPW), 8)
        pltpu.sync_copy(negt_buf, negt_out.at[pl.ds(obase, NNEG * QPW)])

    return body(cw, xw, neg2d, w2_center, w2_context)


def _tc_loss(pos2d, negflat):
    def body(pos_ref, neg_ref, out_ref):
        def log_sigmoid(x):
            return jnp.minimum(x, 0.0) - jnp.log(1.0 + jnp.exp(-jnp.abs(x)))
        s = jnp.sum(log_sigmoid(pos_ref[...])) \
            + jnp.sum(log_sigmoid(-neg_ref[...]))
        out_ref[0, 0] = -s / B

    return pl.pallas_call(
        body,
        out_shape=jax.ShapeDtypeStruct((1, 1), jnp.float32),
        out_specs=pl.BlockSpec(memory_space=pltpu.SMEM),
    )(pos2d, negflat)


def kernel(center_words, context_words, negative_words, W_center, W_context):
    cw = center_words.astype(jnp.int32)
    xw = context_words.astype(jnp.int32)
    neg2d = (negative_words.astype(jnp.int32)
             .reshape(NW, NCHUNK, CHUNK, NNEG)
             .transpose(0, 1, 3, 2)
             .reshape(B * NNEG // NIDX_W, NIDX_W))
    pos, negt = _sc_scores(cw, xw, neg2d, W_center, W_context)
    loss = _tc_loss(pos.reshape(B // NIDX_W, NIDX_W), negt)
    return loss[0, 0]
